# Initial kernel scaffold; baseline (speedup 1.0000x reference)
import jax
import jax.numpy as jnp
from jax.experimental import pallas as pl


def kernel(x, edge_index, weights):
    raise NotImplementedError("write your pallas kernel here")


# speedup vs baseline: 17.5406x; 17.5406x over previous
import functools

import jax
import jax.numpy as jnp
from jax import lax
from jax.experimental import pallas as pl
from jax.experimental.pallas import tpu as pltpu
from jax.experimental.pallas import tpu_sc as plsc

SRC_SIZE = 16384
DST_SIZE = 16384
D = 256
DCH = 64
NDC = D // DCH
NC = 2
NS = 16
K = 64
CB = 24


@functools.lru_cache(maxsize=None)
def _make_sc_call(B, E):
    n_phases = B * NDC
    phases_per_core = n_phases // NC
    per_tile = -(-E // NS)
    nblk = -(-per_tile // (K * CB))
    nch = nblk * CB
    rows_per_tile = DST_SIZE // NS

    mesh = plsc.VectorSubcoreMesh(
        core_axis_name="c", subcore_axis_name="s", num_cores=NC, num_subcores=NS
    )

    @functools.partial(
        pl.kernel,
        out_type=jax.ShapeDtypeStruct((n_phases * DST_SIZE, DCH), jnp.float32),
        mesh=mesh,
        scratch_types=[
            pltpu.VMEM((CB, K), jnp.int32),
            pltpu.VMEM((CB, K), jnp.int32),
            pltpu.VMEM((CB, K), jnp.float32),
            pltpu.VMEM((K,), jnp.int32),
            pltpu.VMEM((DST_SIZE,), jnp.float32),
            pltpu.VMEM((K, DCH), jnp.float32),
            pltpu.VMEM_SHARED((DST_SIZE, DCH), jnp.float32),
            pltpu.SemaphoreType.DMA,
        ],
        compiler_params=pltpu.CompilerParams(
            needs_layout_passes=False, use_tc_tiling_on_sc=False
        ),
    )
    def proj(x_hbm, src_hbm, dst_hbm, w_hbm, out_hbm,
             src_b, dst_b, w_b, idx_k, norm_v, rows_v, acc, sem):
        c = lax.axis_index("c")
        s = lax.axis_index("s")
        zf = jnp.zeros((16,), jnp.float32)

        def zn(i, _):
            norm_v[pl.ds(i * 16, 16)] = zf
            return 0
        lax.fori_loop(0, DST_SIZE // 16, zn, 0)

        for t in range(NS):
            def na(b, _):
                pltpu.sync_copy(dst_hbm.at[t, pl.ds(b * CB, CB)], dst_b)
                pltpu.sync_copy(w_hbm.at[t, pl.ds(b * CB, CB)], w_b)

                def nb(j, _):
                    for kk in range(K // 16):
                        dd = dst_b[j, pl.ds(kk * 16, 16)]
                        ww = w_b[j, pl.ds(kk * 16, 16)]
                        plsc.addupdate_scatter(norm_v, [dd], ww)
                    return 0
                lax.fori_loop(0, CB, nb, 0)
                return 0
            lax.fori_loop(0, nblk, na, 0)

        for i in range(phases_per_core):
            p = c * phases_per_core + i
            base = p * SRC_SIZE

            def zr(r, _):
                for kk in range(DCH // 16):
                    rows_v[r, pl.ds(kk * 16, 16)] = zf
                return 0
            lax.fori_loop(0, K, zr, 0)
            for z in range(rows_per_tile // K):
                pltpu.sync_copy(
                    rows_v, acc.at[pl.ds(s * rows_per_tile + z * K, K)]
                )

            plsc.subcore_barrier()

            def blk(b, _):
                pltpu.sync_copy(src_hbm.at[s, pl.ds(b * CB, CB)], src_b)
                pltpu.sync_copy(dst_hbm.at[s, pl.ds(b * CB, CB)], dst_b)
                pltpu.sync_copy(w_hbm.at[s, pl.ds(b * CB, CB)], w_b)

                def mb(j, _):
                    for kk in range(K // 16):
                        idx_k[pl.ds(kk * 16, 16)] = (
                            src_b[j, pl.ds(kk * 16, 16)] + base
                        )
                    pltpu.async_copy(x_hbm.at[idx_k], rows_v, sem).wait()
                    for g in range(K // 16):
                        dd = dst_b[j, pl.ds(g * 16, 16)]
                        nn = plsc.load_gather(norm_v, [dd])
                        ww = w_b[j, pl.ds(g * 16, 16)] / (nn + 1e-8)
                        for e in range(16):
                            ws = ww[e]
                            for kk in range(DCH // 16):
                                rows_v[g * 16 + e, pl.ds(kk * 16, 16)] = (
                                    rows_v[g * 16 + e, pl.ds(kk * 16, 16)] * ws
                                )
                    pltpu.sync_copy(rows_v, acc.at[dst_b.at[j]], add=True)
                    return 0
                lax.fori_loop(0, CB, mb, 0)
                return 0
            lax.fori_loop(0, nblk, blk, 0)

            plsc.subcore_barrier()

            pltpu.sync_copy(
                acc.at[pl.ds(s * rows_per_tile, rows_per_tile)],
                out_hbm.at[pl.ds(p * DST_SIZE + s * rows_per_tile, rows_per_tile)],
            )
            if i + 1 < phases_per_core:
                plsc.subcore_barrier()

    return proj, nch


@jax.jit
def kernel(x, edge_index, weights):
    B = x.shape[0]
    E = weights.shape[0]
    proj, nch = _make_sc_call(B, E)
    epad = NS * nch * K
    pad = epad - E
    src_p = jnp.pad(edge_index[0], (0, pad)).reshape(NS, nch, K)
    dst_p = jnp.pad(edge_index[1], (0, pad)).reshape(NS, nch, K)
    w_p = jnp.pad(weights, (0, pad)).reshape(NS, nch, K)
    x_r = (
        x.reshape(B, SRC_SIZE, NDC, DCH)
        .transpose(0, 2, 1, 3)
        .reshape(B * NDC * SRC_SIZE, DCH)
    )
    out = proj(x_r, src_p, dst_p, w_p)
    out = (
        out.reshape(B, NDC, DST_SIZE, DCH)
        .transpose(0, 2, 1, 3)
        .reshape(B, DST_SIZE, D)
    )
    return out


# speedup vs baseline: 34.2917x; 1.9550x over previous
import functools

import jax
import jax.numpy as jnp
from jax import lax
from jax.experimental import pallas as pl
from jax.experimental.pallas import tpu as pltpu
from jax.experimental.pallas import tpu_sc as plsc

SRC_SIZE = 16384
DST_SIZE = 16384
D = 256
DCH = 64
NDC = D // DCH
NC = 2
NS = 16
K = 64
CB = 24


@functools.lru_cache(maxsize=None)
def _make_sc_call(B, E):
    n_phases = B * NDC
    phases_per_core = n_phases // NC
    per_tile = -(-E // NS)
    nblk = -(-per_tile // (K * CB))
    nch = nblk * CB
    rows_per_tile = DST_SIZE // NS

    mesh = plsc.VectorSubcoreMesh(
        core_axis_name="c", subcore_axis_name="s", num_cores=NC, num_subcores=NS
    )

    @functools.partial(
        pl.kernel,
        out_type=jax.ShapeDtypeStruct((n_phases * DST_SIZE, DCH), jnp.float32),
        mesh=mesh,
        scratch_types=[
            pltpu.VMEM((CB, K), jnp.int32),
            pltpu.VMEM((CB, K), jnp.int32),
            pltpu.VMEM((CB, K), jnp.float32),
            pltpu.VMEM((nch, K), jnp.float32),
            pltpu.VMEM((2, K), jnp.int32),
            pltpu.VMEM((1024, 16), jnp.float32),
            pltpu.VMEM((2, K, DCH), jnp.float32),
            pltpu.VMEM((2, K, DCH), jnp.float32),
            pltpu.VMEM((64, 16), jnp.float32),
            pltpu.VMEM((128,), jnp.int32),
            pltpu.VMEM_SHARED((1024, 16), jnp.float32),
            pltpu.VMEM_SHARED((DST_SIZE, DCH), jnp.float32),
            pltpu.SemaphoreType.DMA,
            pltpu.SemaphoreType.DMA,
            pltpu.SemaphoreType.DMA,
            pltpu.SemaphoreType.DMA,
        ],
        compiler_params=pltpu.CompilerParams(
            needs_layout_passes=False, use_tc_tiling_on_sc=False
        ),
    )
    def proj(x_hbm, src_hbm, dst_hbm, w_hbm, out_hbm,
             src_b, dst_b, w_b, wn_full, idx2, norm_v, grows, srows,
             zbuf, ridx, norm_sh, acc, gsem0, gsem1, ssem0, ssem1):
        c = lax.axis_index("c")
        s = lax.axis_index("s")
        zf = jnp.zeros((16,), jnp.float32)
        it = lax.iota(jnp.int32, 16)
        gsems = (gsem0, gsem1)
        ssems = (ssem0, ssem1)

        def zn(i, _):
            norm_v[i, pl.ds(0, 16)] = zf
            return 0
        lax.fori_loop(0, 1024, zn, 0)
        for r in range(64):
            zbuf[r, pl.ds(0, 16)] = zf
        pltpu.sync_copy(zbuf, norm_sh.at[pl.ds(s * 64, 64)])

        def na(bk, _):
            pltpu.sync_copy(dst_hbm.at[s, pl.ds(bk * CB, CB)], dst_b)
            pltpu.sync_copy(w_hbm.at[s, pl.ds(bk * CB, CB)], w_b)

            def nb(j, _):
                for g in range(K // 16):
                    dd = dst_b[j, pl.ds(g * 16, 16)]
                    ww = w_b[j, pl.ds(g * 16, 16)]
                    plsc.addupdate_scatter(norm_v, [dd >> 4, dd & 15], ww)
                return 0
            lax.fori_loop(0, CB, nb, 0)
            return 0
        lax.fori_loop(0, nblk, na, 0)

        plsc.subcore_barrier()
        for r in range(8):
            for q in range(8):
                ridx[pl.ds(q * 16, 16)] = it + (r * 128 + q * 16)
            pltpu.sync_copy(
                norm_v.at[pl.ds(r * 128, 128)], norm_sh.at[ridx], add=True
            )
        plsc.subcore_barrier()
        pltpu.sync_copy(norm_sh, norm_v)

        def wa(bk, _):
            pltpu.sync_copy(dst_hbm.at[s, pl.ds(bk * CB, CB)], dst_b)
            pltpu.sync_copy(w_hbm.at[s, pl.ds(bk * CB, CB)], w_b)

            def wb(j, _):
                for g in range(K // 16):
                    dd = dst_b[j, pl.ds(g * 16, 16)]
                    nn = plsc.load_gather(norm_v, [dd >> 4, dd & 15])
                    wn_full[bk * CB + j, pl.ds(g * 16, 16)] = (
                        w_b[j, pl.ds(g * 16, 16)] / (nn + 1e-8)
                    )
                return 0
            lax.fori_loop(0, CB, wb, 0)
            return 0
        lax.fori_loop(0, nblk, wa, 0)

        for i in range(phases_per_core):
            p = c * phases_per_core + i
            base = p * SRC_SIZE

            def zr(r, _):
                for kk in range(DCH // 16):
                    grows[0, r, pl.ds(kk * 16, 16)] = zf
                return 0
            lax.fori_loop(0, K, zr, 0)
            for z in range(rows_per_tile // K):
                pltpu.sync_copy(
                    grows.at[0], acc.at[pl.ds(s * rows_per_tile + z * K, K)]
                )

            plsc.subcore_barrier()

            def blk(bk, _):
                pltpu.sync_copy(src_hbm.at[s, pl.ds(bk * CB, CB)], src_b)
                pltpu.sync_copy(dst_hbm.at[s, pl.ds(bk * CB, CB)], dst_b)
                for b in range(2):
                    for kk in range(K // 16):
                        idx2[b, pl.ds(kk * 16, 16)] = (
                            src_b[b, pl.ds(kk * 16, 16)] + base
                        )
                    pltpu.async_copy(x_hbm.at[idx2.at[b]], grows.at[b], gsems[b])

                def pair(tt, _):
                    for b in range(2):
                        j = tt * 2 + b
                        pltpu.make_async_copy(
                            x_hbm.at[idx2.at[b]], grows.at[b], gsems[b]
                        ).wait()

                        @pl.when(j >= 2)
                        def _():
                            pltpu.make_async_copy(
                                srows.at[b], acc.at[dst_b.at[j]], ssems[b]
                            ).wait()

                        wrow = bk * CB + j
                        for g in range(K // 16):
                            ww = wn_full[wrow, pl.ds(g * 16, 16)]
                            for e in range(16):
                                ws = ww[e]
                                r0 = g * 16 + e
                                for kk in range(DCH // 16):
                                    srows[b, r0, pl.ds(kk * 16, 16)] = (
                                        grows[b, r0, pl.ds(kk * 16, 16)] * ws
                                    )
                        pltpu.async_copy(
                            srows.at[b], acc.at[dst_b.at[j]], ssems[b], add=True
                        )

                        @pl.when(j + 2 < CB)
                        def _():
                            for kk in range(K // 16):
                                idx2[b, pl.ds(kk * 16, 16)] = (
                                    src_b[j + 2, pl.ds(kk * 16, 16)] + base
                                )
                            pltpu.async_copy(
                                x_hbm.at[idx2.at[b]], grows.at[b], gsems[b]
                            )
                    return 0
                lax.fori_loop(0, CB // 2, pair, 0)
                for b in range(2):
                    pltpu.make_async_copy(
                        srows.at[b], acc.at[dst_b.at[b]], ssems[b]
                    ).wait()
                return 0
            lax.fori_loop(0, nblk, blk, 0)

            plsc.subcore_barrier()

            pltpu.sync_copy(
                acc.at[pl.ds(s * rows_per_tile, rows_per_tile)],
                out_hbm.at[pl.ds(p * DST_SIZE + s * rows_per_tile, rows_per_tile)],
            )
            if i + 1 < phases_per_core:
                plsc.subcore_barrier()

    return proj, nch


@jax.jit
def kernel(x, edge_index, weights):
    B = x.shape[0]
    E = weights.shape[0]
    proj, nch = _make_sc_call(B, E)
    epad = NS * nch * K
    pad = epad - E
    src_p = jnp.pad(edge_index[0], (0, pad)).reshape(NS, nch, K)
    dst_p = jnp.pad(edge_index[1], (0, pad)).reshape(NS, nch, K)
    w_p = jnp.pad(weights, (0, pad)).reshape(NS, nch, K)
    x_r = (
        x.reshape(B, SRC_SIZE, NDC, DCH)
        .transpose(0, 2, 1, 3)
        .reshape(B * NDC * SRC_SIZE, DCH)
    )
    out = proj(x_r, src_p, dst_p, w_p)
    out = (
        out.reshape(B, NDC, DST_SIZE, DCH)
        .transpose(0, 2, 1, 3)
        .reshape(B, DST_SIZE, D)
    )
    return out


# speedup vs baseline: 37.8598x; 1.1041x over previous
import functools

import jax
import jax.numpy as jnp
from jax import lax
from jax.experimental import pallas as pl
from jax.experimental.pallas import tpu as pltpu
from jax.experimental.pallas import tpu_sc as plsc

SRC_SIZE = 16384
DST_SIZE = 16384
D = 256
DCH = 64
NDC = D // DCH
NC = 2
NS = 16
K = 64
CB = 24


@functools.lru_cache(maxsize=None)
def _make_sc_call(B, E):
    n_phases = B * NDC
    phases_per_core = n_phases // NC
    per_tile = -(-E // NS)
    nblk = -(-per_tile // (K * CB))
    nch = nblk * CB
    rows_per_tile = DST_SIZE // NS

    mesh = plsc.VectorSubcoreMesh(
        core_axis_name="c", subcore_axis_name="s", num_cores=NC, num_subcores=NS
    )

    @functools.partial(
        pl.kernel,
        out_type=(
            jax.ShapeDtypeStruct((B * DST_SIZE, D), jnp.float32),
            jax.ShapeDtypeStruct((n_phases * SRC_SIZE, DCH), jnp.float32),
        ),
        mesh=mesh,
        scratch_types=[
            pltpu.VMEM((CB, K), jnp.int32),
            pltpu.VMEM((CB, K), jnp.int32),
            pltpu.VMEM((CB, K), jnp.float32),
            pltpu.VMEM((nch, K), jnp.float32),
            pltpu.VMEM((2, K), jnp.int32),
            pltpu.VMEM((1024, 16), jnp.float32),
            pltpu.VMEM((2, K, DCH), jnp.float32),
            pltpu.VMEM((2, K, DCH), jnp.float32),
            pltpu.VMEM((64, 16), jnp.float32),
            pltpu.VMEM((128,), jnp.int32),
            pltpu.VMEM_SHARED((1024, 16), jnp.float32),
            pltpu.VMEM_SHARED((DST_SIZE, DCH), jnp.float32),
            pltpu.SemaphoreType.DMA,
            pltpu.SemaphoreType.DMA,
            pltpu.SemaphoreType.DMA,
            pltpu.SemaphoreType.DMA,
        ],
        compiler_params=pltpu.CompilerParams(
            needs_layout_passes=False, use_tc_tiling_on_sc=False
        ),
    )
    def proj(x_hbm, src_hbm, dst_hbm, w_hbm, out_hbm, xt_hbm,
             src_b, dst_b, w_b, wn_full, idx2, norm_v, grows, srows,
             zbuf, ridx, norm_sh, acc, gsem0, gsem1, ssem0, ssem1):
        c = lax.axis_index("c")
        s = lax.axis_index("s")
        zf = jnp.zeros((16,), jnp.float32)
        it = lax.iota(jnp.int32, 16)
        gsems = (gsem0, gsem1)
        ssems = (ssem0, ssem1)

        n_tp = NDC * 16
        xrow0 = c * SRC_SIZE + s * (SRC_SIZE // NS)

        def tp_read(k, b):
            pltpu.async_copy(
                x_hbm.at[
                    pl.ds(xrow0 + (k & 15) * K, K),
                    pl.ds((k >> 4) * DCH, DCH),
                ],
                grows.at[b],
                gsems[b],
            )

        for b in range(2):
            tp_read(b, b)

        def tp(kk, _):
            for b in range(2):
                k = kk * 2 + b
                pltpu.make_async_copy(
                    x_hbm.at[pl.ds(0, K), pl.ds(0, DCH)], grows.at[b], gsems[b]
                ).wait()

                @pl.when(k >= 2)
                def _():
                    pltpu.make_async_copy(
                        xt_hbm.at[pl.ds(0, K)], grows.at[b], ssems[b]
                    ).wait()

                pltpu.async_copy(grows.at[b], xt_hbm.at[pl.ds(
                    (c * NDC + (k >> 4)) * SRC_SIZE
                    + s * (SRC_SIZE // NS) + (k & 15) * K, K)], ssems[b])

                @pl.when(k + 2 < n_tp)
                def _():
                    k2 = k + 2
                    pltpu.async_copy(
                        x_hbm.at[
                            pl.ds(xrow0 + (k2 & 15) * K, K),
                            pl.ds((k2 >> 4) * DCH, DCH),
                        ],
                        grows.at[b],
                        gsems[b],
                    )
            return 0
        lax.fori_loop(0, n_tp // 2, tp, 0)
        for b in range(2):
            pltpu.make_async_copy(
                xt_hbm.at[pl.ds(0, K)], grows.at[b], ssems[b]
            ).wait()

        def zn(i, _):
            norm_v[i, pl.ds(0, 16)] = zf
            return 0
        lax.fori_loop(0, 1024, zn, 0)
        for r in range(64):
            zbuf[r, pl.ds(0, 16)] = zf
        pltpu.sync_copy(zbuf, norm_sh.at[pl.ds(s * 64, 64)])

        def na(bk, _):
            pltpu.sync_copy(dst_hbm.at[s, pl.ds(bk * CB, CB)], dst_b)
            pltpu.sync_copy(w_hbm.at[s, pl.ds(bk * CB, CB)], w_b)

            def nb(j, _):
                for g in range(K // 16):
                    dd = dst_b[j, pl.ds(g * 16, 16)]
                    ww = w_b[j, pl.ds(g * 16, 16)]
                    plsc.addupdate_scatter(norm_v, [dd >> 4, dd & 15], ww)
                return 0
            lax.fori_loop(0, CB, nb, 0)
            return 0
        lax.fori_loop(0, nblk, na, 0)

        plsc.subcore_barrier()
        for r in range(8):
            for q in range(8):
                ridx[pl.ds(q * 16, 16)] = it + (r * 128 + q * 16)
            pltpu.sync_copy(
                norm_v.at[pl.ds(r * 128, 128)], norm_sh.at[ridx], add=True
            )
        plsc.subcore_barrier()
        pltpu.sync_copy(norm_sh, norm_v)

        def wa(bk, _):
            pltpu.sync_copy(dst_hbm.at[s, pl.ds(bk * CB, CB)], dst_b)
            pltpu.sync_copy(w_hbm.at[s, pl.ds(bk * CB, CB)], w_b)

            def wb(j, _):
                for g in range(K // 16):
                    dd = dst_b[j, pl.ds(g * 16, 16)]
                    nn = plsc.load_gather(norm_v, [dd >> 4, dd & 15])
                    wn_full[bk * CB + j, pl.ds(g * 16, 16)] = (
                        w_b[j, pl.ds(g * 16, 16)] / (nn + 1e-8)
                    )
                return 0
            lax.fori_loop(0, CB, wb, 0)
            return 0
        lax.fori_loop(0, nblk, wa, 0)

        for i in range(phases_per_core):
            p = c * phases_per_core + i
            base = p * SRC_SIZE

            def zr(r, _):
                for kk in range(DCH // 16):
                    grows[0, r, pl.ds(kk * 16, 16)] = zf
                return 0
            lax.fori_loop(0, K, zr, 0)
            for z in range(rows_per_tile // K):
                pltpu.sync_copy(
                    grows.at[0], acc.at[pl.ds(s * rows_per_tile + z * K, K)]
                )

            plsc.subcore_barrier()

            def blk(bk, _):
                pltpu.sync_copy(src_hbm.at[s, pl.ds(bk * CB, CB)], src_b)
                pltpu.sync_copy(dst_hbm.at[s, pl.ds(bk * CB, CB)], dst_b)
                for b in range(2):
                    for kk in range(K // 16):
                        idx2[b, pl.ds(kk * 16, 16)] = (
                            src_b[b, pl.ds(kk * 16, 16)] + base
                        )
                    pltpu.async_copy(xt_hbm.at[idx2.at[b]], grows.at[b], gsems[b])

                def pair(tt, _):
                    for b in range(2):
                        j = tt * 2 + b
                        pltpu.make_async_copy(
                            xt_hbm.at[idx2.at[b]], grows.at[b], gsems[b]
                        ).wait()

                        @pl.when(j >= 2)
                        def _():
                            pltpu.make_async_copy(
                                srows.at[b], acc.at[dst_b.at[j]], ssems[b]
                            ).wait()

                        wrow = bk * CB + j
                        for g in range(K // 16):
                            ww = wn_full[wrow, pl.ds(g * 16, 16)]
                            for e in range(16):
                                ws = ww[e]
                                r0 = g * 16 + e
                                for kk in range(DCH // 16):
                                    srows[b, r0, pl.ds(kk * 16, 16)] = (
                                        grows[b, r0, pl.ds(kk * 16, 16)] * ws
                                    )
                        pltpu.async_copy(
                            srows.at[b], acc.at[dst_b.at[j]], ssems[b], add=True
                        )

                        @pl.when(j + 2 < CB)
                        def _():
                            for kk in range(K // 16):
                                idx2[b, pl.ds(kk * 16, 16)] = (
                                    src_b[j + 2, pl.ds(kk * 16, 16)] + base
                                )
                            pltpu.async_copy(
                                xt_hbm.at[idx2.at[b]], grows.at[b], gsems[b]
                            )
                    return 0
                lax.fori_loop(0, CB // 2, pair, 0)
                for b in range(2):
                    pltpu.make_async_copy(
                        srows.at[b], acc.at[dst_b.at[b]], ssems[b]
                    ).wait()
                return 0
            lax.fori_loop(0, nblk, blk, 0)

            plsc.subcore_barrier()

            pltpu.sync_copy(
                acc.at[pl.ds(s * rows_per_tile, rows_per_tile)],
                out_hbm.at[
                    pl.ds(c * DST_SIZE + s * rows_per_tile, rows_per_tile),
                    pl.ds(i * DCH, DCH),
                ],
            )
            if i + 1 < phases_per_core:
                plsc.subcore_barrier()

    return proj, nch


@jax.jit
def kernel(x, edge_index, weights):
    B = x.shape[0]
    E = weights.shape[0]
    proj, nch = _make_sc_call(B, E)
    epad = NS * nch * K
    pad = epad - E
    src_p = jnp.pad(edge_index[0], (0, pad)).reshape(NS, nch, K)
    dst_p = jnp.pad(edge_index[1], (0, pad)).reshape(NS, nch, K)
    w_p = jnp.pad(weights, (0, pad)).reshape(NS, nch, K)
    x_r = x.reshape(B * SRC_SIZE, D)
    out, _ = proj(x_r, src_p, dst_p, w_p)
    return out.reshape(B, DST_SIZE, D)


# speedup vs baseline: 44.9610x; 1.1876x over previous
import functools

import jax
import jax.numpy as jnp
from jax import lax
from jax.experimental import pallas as pl
from jax.experimental.pallas import tpu as pltpu
from jax.experimental.pallas import tpu_sc as plsc

SRC_SIZE = 16384
DST_SIZE = 16384
D = 256
DCH = 64
NDC = D // DCH
NC = 2
NS = 16
K = 64
CB = 24


@functools.lru_cache(maxsize=None)
def _make_sc_call(B, E):
    n_phases = B * NDC
    phases_per_core = n_phases // NC
    per_tile = -(-E // NS)
    nblk = -(-per_tile // (K * CB))
    nch = nblk * CB
    rows_per_tile = DST_SIZE // NS

    mesh = plsc.VectorSubcoreMesh(
        core_axis_name="c", subcore_axis_name="s", num_cores=NC, num_subcores=NS
    )

    @functools.partial(
        pl.kernel,
        out_type=(
            jax.ShapeDtypeStruct((B * DST_SIZE, D), jnp.float32),
            jax.ShapeDtypeStruct((n_phases * SRC_SIZE, DCH), jnp.float32),
        ),
        mesh=mesh,
        scratch_types=[
            pltpu.VMEM((CB, K), jnp.int32),
            pltpu.VMEM((CB, K), jnp.int32),
            pltpu.VMEM((CB, K), jnp.float32),
            pltpu.VMEM((nch, K), jnp.float32),
            pltpu.VMEM((4, K), jnp.int32),
            pltpu.VMEM((1024, 16), jnp.float32),
            pltpu.VMEM((4, K, DCH), jnp.float32),
            pltpu.VMEM((2, K, DCH), jnp.float32),
            pltpu.VMEM((64, 16), jnp.float32),
            pltpu.VMEM((128,), jnp.int32),
            pltpu.VMEM_SHARED((1024, 16), jnp.float32),
            pltpu.VMEM_SHARED((DST_SIZE, DCH), jnp.float32),
            pltpu.SemaphoreType.DMA,
            pltpu.SemaphoreType.DMA,
            pltpu.SemaphoreType.DMA,
            pltpu.SemaphoreType.DMA,
            pltpu.SemaphoreType.DMA,
            pltpu.SemaphoreType.DMA,
        ],
        compiler_params=pltpu.CompilerParams(
            needs_layout_passes=False, use_tc_tiling_on_sc=False
        ),
    )
    def proj(x_hbm, src_hbm, dst_hbm, w_hbm, out_hbm, xt_hbm,
             src_b, dst_b, w_b, wn_full, idx4, norm_v, grows, srows,
             zbuf, ridx, norm_sh, acc, gsem0, gsem1, gsem2, gsem3,
             ssem0, ssem1):
        c = lax.axis_index("c")
        s = lax.axis_index("s")
        zf = jnp.zeros((16,), jnp.float32)
        it = lax.iota(jnp.int32, 16)
        gsems = (gsem0, gsem1, gsem2, gsem3)
        ssems = (ssem0, ssem1)

        n_tp = NDC * 16
        xrow0 = c * SRC_SIZE + s * (SRC_SIZE // NS)

        def tp_read(k, b):
            pltpu.async_copy(
                x_hbm.at[
                    pl.ds(xrow0 + (k & 15) * K, K),
                    pl.ds((k >> 4) * DCH, DCH),
                ],
                grows.at[b],
                gsems[b],
            )

        for b in range(2):
            tp_read(b, b)

        def tp(kk, _):
            for b in range(2):
                k = kk * 2 + b
                pltpu.make_async_copy(
                    x_hbm.at[pl.ds(0, K), pl.ds(0, DCH)], grows.at[b], gsems[b]
                ).wait()

                @pl.when(k >= 2)
                def _():
                    pltpu.make_async_copy(
                        xt_hbm.at[pl.ds(0, K)], grows.at[b], ssems[b]
                    ).wait()

                pltpu.async_copy(grows.at[b], xt_hbm.at[pl.ds(
                    (c * NDC + (k >> 4)) * SRC_SIZE
                    + s * (SRC_SIZE // NS) + (k & 15) * K, K)], ssems[b])

                @pl.when(k + 2 < n_tp)
                def _():
                    k2 = k + 2
                    pltpu.async_copy(
                        x_hbm.at[
                            pl.ds(xrow0 + (k2 & 15) * K, K),
                            pl.ds((k2 >> 4) * DCH, DCH),
                        ],
                        grows.at[b],
                        gsems[b],
                    )
            return 0
        lax.fori_loop(0, n_tp // 2, tp, 0)
        for b in range(2):
            pltpu.make_async_copy(
                xt_hbm.at[pl.ds(0, K)], grows.at[b], ssems[b]
            ).wait()

        def zn(i, _):
            norm_v[i, pl.ds(0, 16)] = zf
            return 0
        lax.fori_loop(0, 1024, zn, 0)
        for r in range(64):
            zbuf[r, pl.ds(0, 16)] = zf
        pltpu.sync_copy(zbuf, norm_sh.at[pl.ds(s * 64, 64)])

        def na(bk, _):
            pltpu.sync_copy(dst_hbm.at[s, pl.ds(bk * CB, CB)], dst_b)
            pltpu.sync_copy(w_hbm.at[s, pl.ds(bk * CB, CB)], w_b)

            def nb(j, _):
                for g in range(K // 16):
                    dd = dst_b[j, pl.ds(g * 16, 16)]
                    ww = w_b[j, pl.ds(g * 16, 16)]
                    plsc.addupdate_scatter(norm_v, [dd >> 4, dd & 15], ww)
                return 0
            lax.fori_loop(0, CB, nb, 0)
            return 0
        lax.fori_loop(0, nblk, na, 0)

        plsc.subcore_barrier()
        for r in range(8):
            for q in range(8):
                ridx[pl.ds(q * 16, 16)] = it + (r * 128 + q * 16)
            pltpu.sync_copy(
                norm_v.at[pl.ds(r * 128, 128)], norm_sh.at[ridx], add=True
            )
        plsc.subcore_barrier()
        pltpu.sync_copy(norm_sh, norm_v)

        def wa(bk, _):
            pltpu.sync_copy(dst_hbm.at[s, pl.ds(bk * CB, CB)], dst_b)
            pltpu.sync_copy(w_hbm.at[s, pl.ds(bk * CB, CB)], w_b)

            def wb(j, _):
                for g in range(K // 16):
                    dd = dst_b[j, pl.ds(g * 16, 16)]
                    nn = plsc.load_gather(norm_v, [dd >> 4, dd & 15])
                    wn_full[bk * CB + j, pl.ds(g * 16, 16)] = (
                        w_b[j, pl.ds(g * 16, 16)] / (nn + 1e-8)
                    )
                return 0
            lax.fori_loop(0, CB, wb, 0)
            return 0
        lax.fori_loop(0, nblk, wa, 0)

        for i in range(phases_per_core):
            p = c * phases_per_core + i
            base = p * SRC_SIZE

            def zr(r, _):
                for kk in range(DCH // 16):
                    grows[0, r, pl.ds(kk * 16, 16)] = zf
                return 0
            lax.fori_loop(0, K, zr, 0)
            for z in range(rows_per_tile // K):
                pltpu.sync_copy(
                    grows.at[0], acc.at[pl.ds(s * rows_per_tile + z * K, K)]
                )

            plsc.subcore_barrier()

            def blk(bk, _):
                pltpu.sync_copy(src_hbm.at[s, pl.ds(bk * CB, CB)], src_b)
                pltpu.sync_copy(dst_hbm.at[s, pl.ds(bk * CB, CB)], dst_b)
                for b in range(4):
                    for kk in range(K // 16):
                        idx4[b, pl.ds(kk * 16, 16)] = (
                            src_b[b, pl.ds(kk * 16, 16)] + base
                        )
                    pltpu.async_copy(xt_hbm.at[idx4.at[b]], grows.at[b], gsems[b])

                def quad(tt, _):
                    for b in range(4):
                        j = tt * 4 + b
                        b2 = b % 2
                        pltpu.make_async_copy(
                            xt_hbm.at[idx4.at[b]], grows.at[b], gsems[b]
                        ).wait()

                        @pl.when(j >= 2)
                        def _():
                            pltpu.make_async_copy(
                                srows.at[b2], acc.at[dst_b.at[j]], ssems[b2]
                            ).wait()

                        wrow = bk * CB + j
                        for g in range(K // 16):
                            ww = wn_full[wrow, pl.ds(g * 16, 16)]
                            for e in range(16):
                                ws = ww[e]
                                r0 = g * 16 + e
                                for kk in range(DCH // 16):
                                    srows[b2, r0, pl.ds(kk * 16, 16)] = (
                                        grows[b, r0, pl.ds(kk * 16, 16)] * ws
                                    )
                        pltpu.async_copy(
                            srows.at[b2], acc.at[dst_b.at[j]], ssems[b2], add=True
                        )

                        @pl.when(j + 4 < CB)
                        def _():
                            for kk in range(K // 16):
                                idx4[b, pl.ds(kk * 16, 16)] = (
                                    src_b[j + 4, pl.ds(kk * 16, 16)] + base
                                )
                            pltpu.async_copy(
                                xt_hbm.at[idx4.at[b]], grows.at[b], gsems[b]
                            )
                    return 0
                lax.fori_loop(0, CB // 4, quad, 0)
                for b2 in range(2):
                    pltpu.make_async_copy(
                        srows.at[b2], acc.at[dst_b.at[b2]], ssems[b2]
                    ).wait()
                return 0
            lax.fori_loop(0, nblk, blk, 0)

            plsc.subcore_barrier()

            pltpu.sync_copy(
                acc.at[pl.ds(s * rows_per_tile, rows_per_tile)],
                out_hbm.at[
                    pl.ds(c * DST_SIZE + s * rows_per_tile, rows_per_tile),
                    pl.ds(i * DCH, DCH),
                ],
            )
            if i + 1 < phases_per_core:
                plsc.subcore_barrier()

    return proj, nch


@jax.jit
def kernel(x, edge_index, weights):
    B = x.shape[0]
    E = weights.shape[0]
    proj, nch = _make_sc_call(B, E)
    epad = NS * nch * K
    pad = epad - E
    src_p = jnp.pad(edge_index[0], (0, pad)).reshape(NS, nch, K)
    dst_p = jnp.pad(edge_index[1], (0, pad)).reshape(NS, nch, K)
    w_p = jnp.pad(weights, (0, pad)).reshape(NS, nch, K)
    x_r = x.reshape(B * SRC_SIZE, D)
    out, _ = proj(x_r, src_p, dst_p, w_p)
    return out.reshape(B, DST_SIZE, D)
